# baseline (speedup 1.0000x reference)
import functools

import jax
import jax.numpy as jnp
from jax import lax
from jax.experimental import pallas as pl
from jax.experimental.pallas import tpu as pltpu
from jax.experimental.pallas import tpu_sc as plsc

_N_E = 8192
_D = 32
_ROWS = 256


_WIN = 2048


def _argmin_body(z_ref, e_ref, idx_ref):
    zb = z_ref[...]
    zb_bf = zb.astype(jnp.bfloat16)
    zn = jnp.sum(zb * zb, axis=1, keepdims=True)
    e = e_ref[...]
    acc_v = jnp.full((_ROWS,), jnp.inf, jnp.float32)
    acc_i = jnp.zeros((_ROWS,), jnp.int32)
    for w in range(_N_E // _WIN):
        ew = e[w * _WIN:(w + 1) * _WIN]
        en_w = jnp.sum(ew * ew, axis=1)[None, :]
        dot = lax.dot_general(zb_bf, ew.astype(jnp.bfloat16),
                              (((1,), (1,)), ((), ())),
                              preferred_element_type=jnp.float32)
        dw = (zn + en_w) - 2.0 * dot
        mv = jnp.min(dw, axis=1)
        lane = lax.broadcasted_iota(jnp.int32, (_ROWS, _WIN), 1)
        iv = jnp.min(jnp.where(dw == mv[:, None], lane, jnp.int32(2**31 - 1)),
                     axis=1) + w * _WIN
        take = mv < acc_v
        acc_i = jnp.where(take, iv, acc_i)
        acc_v = jnp.where(take, mv, acc_v).astype(jnp.bfloat16).astype(jnp.float32)
    idx_ref[0, 0, :] = acc_i


def _tc_argmin(zf, emb):
    nblk = zf.shape[0] // _ROWS
    out = pl.pallas_call(
        _argmin_body,
        grid=(nblk,),
        in_specs=[
            pl.BlockSpec((_ROWS, _D), lambda i: (i, 0)),
            pl.BlockSpec((_N_E, _D), lambda i: (0, 0)),
        ],
        out_specs=pl.BlockSpec((1, 1, _ROWS), lambda i: (i, 0, 0)),
        out_shape=jax.ShapeDtypeStruct((nblk, 1, _ROWS), jnp.int32),
    )(zf, emb)
    return out.reshape(-1)


def _sc_gather(emb, idx):
    info = plsc.get_sparse_core_info()
    nc, ns = info.num_cores, info.num_subcores
    nw = nc * ns
    b = idx.shape[0]
    bpw = b // nw

    @functools.partial(
        pl.kernel,
        mesh=plsc.VectorSubcoreMesh(core_axis_name="c", subcore_axis_name="s"),
        compiler_params=pltpu.CompilerParams(use_tc_tiling_on_sc=False),
        out_type=jax.ShapeDtypeStruct((b, _D), jnp.float32),
        scratch_types=[
            pltpu.VMEM((bpw,), jnp.int32),
            pltpu.VMEM((bpw, _D), jnp.float32),
            pltpu.SemaphoreType.DMA,
        ],
    )
    def gk(table_hbm, idx_hbm, out_hbm, idx_v, rows_v, sem):
        wid = lax.axis_index("s") * nc + lax.axis_index("c")
        base = wid * bpw
        pltpu.sync_copy(idx_hbm.at[pl.ds(base, bpw)], idx_v)
        pltpu.async_copy(table_hbm.at[idx_v], rows_v, sem).wait()
        pltpu.sync_copy(rows_v, out_hbm.at[pl.ds(base, bpw)])

    return gk(emb, idx)


def kernel(z, embeddings):
    zf = z.reshape(-1, z.shape[-1])
    idx = _tc_argmin(zf, embeddings)
    zq = _sc_gather(embeddings, idx).reshape(z.shape)
    return zq, idx.reshape(z.shape[:-1])


# speedup vs baseline: 1.2577x; 1.2577x over previous
import functools

import jax
import jax.numpy as jnp
from jax import lax
from jax.experimental import pallas as pl
from jax.experimental.pallas import tpu as pltpu
from jax.experimental.pallas import tpu_sc as plsc

_N_E = 8192
_D = 32
_ROWS = 512
_WIN = 2048


def _argmin_body(z_ref, e_ref, idx_ref):
    zb = z_ref[...]
    zb_bf = zb.astype(jnp.bfloat16)
    zn = jnp.sum(zb * zb, axis=1, keepdims=True)
    e = e_ref[...]
    e_bf2 = e.astype(jnp.bfloat16) * jnp.bfloat16(-2)
    en = jnp.sum(e * e, axis=1)
    acc_v = jnp.full((_ROWS,), jnp.inf, jnp.float32)
    acc_i = jnp.zeros((_ROWS,), jnp.int32)
    big = jnp.int32(2**31 - 1)
    for w in range(_N_E // _WIN):
        ew2 = e_bf2[w * _WIN:(w + 1) * _WIN]
        en_w = en[w * _WIN:(w + 1) * _WIN][None, :]
        dot = lax.dot_general(zb_bf, ew2, (((1,), (1,)), ((), ())),
                              preferred_element_type=jnp.float32)
        dw = (zn + en_w) + dot
        mv = jnp.min(dw, axis=1)
        lane = lax.broadcasted_iota(jnp.int32, (_ROWS, _WIN), 1)
        iv = jnp.min(jnp.where(dw == mv[:, None], lane, big),
                     axis=1) + w * _WIN
        take = mv < acc_v
        acc_i = jnp.where(take, iv, acc_i)
        acc_v = jnp.where(take, mv, acc_v).astype(jnp.bfloat16).astype(jnp.float32)
    idx_ref[0, 0, :] = acc_i


def _tc_argmin(zf, emb):
    nblk = zf.shape[0] // _ROWS
    out = pl.pallas_call(
        _argmin_body,
        grid=(nblk,),
        in_specs=[
            pl.BlockSpec((_ROWS, _D), lambda i: (i, 0)),
            pl.BlockSpec((_N_E, _D), lambda i: (0, 0)),
        ],
        out_specs=pl.BlockSpec((1, 1, _ROWS), lambda i: (i, 0, 0)),
        out_shape=jax.ShapeDtypeStruct((nblk, 1, _ROWS), jnp.int32),
    )(zf, emb)
    return out.reshape(-1)


def _sc_gather(emb, idx):
    info = plsc.get_sparse_core_info()
    nc, ns = info.num_cores, info.num_subcores
    nw = nc * ns
    b = idx.shape[0]
    bpw = b // nw

    @functools.partial(
        pl.kernel,
        mesh=plsc.VectorSubcoreMesh(core_axis_name="c", subcore_axis_name="s"),
        compiler_params=pltpu.CompilerParams(use_tc_tiling_on_sc=False),
        out_type=jax.ShapeDtypeStruct((b, _D), jnp.float32),
        scratch_types=[
            pltpu.VMEM((bpw,), jnp.int32),
            pltpu.VMEM((bpw, _D), jnp.float32),
            pltpu.SemaphoreType.DMA,
        ],
    )
    def gk(table_hbm, idx_hbm, out_hbm, idx_v, rows_v, sem):
        wid = lax.axis_index("s") * nc + lax.axis_index("c")
        base = wid * bpw
        pltpu.sync_copy(idx_hbm.at[pl.ds(base, bpw)], idx_v)
        pltpu.async_copy(table_hbm.at[idx_v], rows_v, sem).wait()
        pltpu.sync_copy(rows_v, out_hbm.at[pl.ds(base, bpw)])

    return gk(emb, idx)


def kernel(z, embeddings):
    zf = z.reshape(-1, z.shape[-1])
    idx = _tc_argmin(zf, embeddings)
    zq = _sc_gather(embeddings, idx).reshape(z.shape)
    return zq, idx.reshape(z.shape[:-1])


# speedup vs baseline: 1.3298x; 1.0573x over previous
import functools

import jax
import jax.numpy as jnp
from jax import lax
from jax.experimental import pallas as pl
from jax.experimental.pallas import tpu as pltpu
from jax.experimental.pallas import tpu_sc as plsc

_N_E = 8192
_D = 32
_ROWS = 512
_WIN = 2048


def _argmin_body(z_ref, e_ref, idx_ref, e2_ref, en_ref):
    @pl.when(pl.program_id(0) == 0)
    def _init():
        e = e_ref[...]
        e2_ref[...] = e.astype(jnp.bfloat16) * jnp.bfloat16(-2)
        en_ref[...] = jnp.sum(e * e, axis=1)[None, :]

    zb = z_ref[...]
    zb_bf = zb.astype(jnp.bfloat16)
    zn = jnp.sum(zb * zb, axis=1, keepdims=True)
    acc_v = jnp.full((_ROWS,), jnp.inf, jnp.float32)
    acc_i = jnp.zeros((_ROWS,), jnp.int32)
    big = jnp.int32(2**31 - 1)
    for w in range(_N_E // _WIN):
        ew2 = e2_ref[w * _WIN:(w + 1) * _WIN, :]
        en_w = en_ref[0:1, w * _WIN:(w + 1) * _WIN]
        dot = lax.dot_general(zb_bf, ew2, (((1,), (1,)), ((), ())),
                              preferred_element_type=jnp.float32)
        dw = (zn + en_w) + dot
        mv = jnp.min(dw, axis=1)
        lane = lax.broadcasted_iota(jnp.int32, (_ROWS, _WIN), 1)
        iv = jnp.min(jnp.where(dw == mv[:, None], lane, big),
                     axis=1) + w * _WIN
        take = mv < acc_v
        acc_i = jnp.where(take, iv, acc_i)
        acc_v = jnp.where(take, mv, acc_v).astype(jnp.bfloat16).astype(jnp.float32)
    idx_ref[0, 0, :] = acc_i


def _tc_argmin(zf, emb):
    nblk = zf.shape[0] // _ROWS
    out = pl.pallas_call(
        _argmin_body,
        grid=(nblk,),
        in_specs=[
            pl.BlockSpec((_ROWS, _D), lambda i: (i, 0)),
            pl.BlockSpec((_N_E, _D), lambda i: (0, 0)),
        ],
        out_specs=pl.BlockSpec((1, 1, _ROWS), lambda i: (i, 0, 0)),
        out_shape=jax.ShapeDtypeStruct((nblk, 1, _ROWS), jnp.int32),
        scratch_shapes=[
            pltpu.VMEM((_N_E, _D), jnp.bfloat16),
            pltpu.VMEM((1, _N_E), jnp.float32),
        ],
    )(zf, emb)
    return out.reshape(-1)


def _sc_gather(emb, idx):
    info = plsc.get_sparse_core_info()
    nc, ns = info.num_cores, info.num_subcores
    nw = nc * ns
    b = idx.shape[0]
    bpw = b // nw

    @functools.partial(
        pl.kernel,
        mesh=plsc.VectorSubcoreMesh(core_axis_name="c", subcore_axis_name="s"),
        compiler_params=pltpu.CompilerParams(use_tc_tiling_on_sc=False),
        out_type=jax.ShapeDtypeStruct((b, _D), jnp.float32),
        scratch_types=[
            pltpu.VMEM((bpw,), jnp.int32),
            pltpu.VMEM((bpw, _D), jnp.float32),
            pltpu.SemaphoreType.DMA,
        ],
    )
    def gk(table_hbm, idx_hbm, out_hbm, idx_v, rows_v, sem):
        wid = lax.axis_index("s") * nc + lax.axis_index("c")
        base = wid * bpw
        pltpu.sync_copy(idx_hbm.at[pl.ds(base, bpw)], idx_v)
        pltpu.async_copy(table_hbm.at[idx_v], rows_v, sem).wait()
        pltpu.sync_copy(rows_v, out_hbm.at[pl.ds(base, bpw)])

    return gk(emb, idx)


def kernel(z, embeddings):
    zf = z.reshape(-1, z.shape[-1])
    idx = _tc_argmin(zf, embeddings)
    zq = _sc_gather(embeddings, idx).reshape(z.shape)
    return zq, idx.reshape(z.shape[:-1])


# speedup vs baseline: 1.3317x; 1.0015x over previous
import functools

import jax
import jax.numpy as jnp
from jax import lax
from jax.experimental import pallas as pl
from jax.experimental.pallas import tpu as pltpu
from jax.experimental.pallas import tpu_sc as plsc

_N_E = 8192
_D = 32
_ROWS = 1024
_WIN = 2048


def _argmin_body(z_ref, e_ref, idx_ref, e2_ref, en_ref):
    @pl.when(pl.program_id(0) == 0)
    def _init():
        e = e_ref[...]
        e2_ref[...] = e.astype(jnp.bfloat16) * jnp.bfloat16(-2)
        en_ref[...] = jnp.sum(e * e, axis=1)[None, :]

    zb = z_ref[...]
    zb_bf = zb.astype(jnp.bfloat16)
    zn = jnp.sum(zb * zb, axis=1, keepdims=True)
    acc_v = jnp.full((_ROWS,), jnp.inf, jnp.float32)
    acc_i = jnp.zeros((_ROWS,), jnp.int32)
    big = jnp.int32(2**31 - 1)
    for w in range(_N_E // _WIN):
        ew2 = e2_ref[w * _WIN:(w + 1) * _WIN, :]
        en_w = en_ref[0:1, w * _WIN:(w + 1) * _WIN]
        dot = lax.dot_general(zb_bf, ew2, (((1,), (1,)), ((), ())),
                              preferred_element_type=jnp.float32)
        dw = (zn + en_w) + dot
        mv = jnp.min(dw, axis=1)
        lane = lax.broadcasted_iota(jnp.int32, (_ROWS, _WIN), 1)
        iv = jnp.min(jnp.where(dw == mv[:, None], lane, big),
                     axis=1) + w * _WIN
        take = mv < acc_v
        acc_i = jnp.where(take, iv, acc_i)
        acc_v = jnp.where(take, mv, acc_v).astype(jnp.bfloat16).astype(jnp.float32)
    idx_ref[0, 0, :] = acc_i


def _tc_argmin(zf, emb):
    nblk = zf.shape[0] // _ROWS
    out = pl.pallas_call(
        _argmin_body,
        grid=(nblk,),
        in_specs=[
            pl.BlockSpec((_ROWS, _D), lambda i: (i, 0)),
            pl.BlockSpec((_N_E, _D), lambda i: (0, 0)),
        ],
        out_specs=pl.BlockSpec((1, 1, _ROWS), lambda i: (i, 0, 0)),
        out_shape=jax.ShapeDtypeStruct((nblk, 1, _ROWS), jnp.int32),
        scratch_shapes=[
            pltpu.VMEM((_N_E, _D), jnp.bfloat16),
            pltpu.VMEM((1, _N_E), jnp.float32),
        ],
    )(zf, emb)
    return out.reshape(-1)


def _sc_gather(emb, idx):
    info = plsc.get_sparse_core_info()
    nc, ns = info.num_cores, info.num_subcores
    nw = nc * ns
    b = idx.shape[0]
    bpw = b // nw

    @functools.partial(
        pl.kernel,
        mesh=plsc.VectorSubcoreMesh(core_axis_name="c", subcore_axis_name="s"),
        compiler_params=pltpu.CompilerParams(use_tc_tiling_on_sc=False),
        out_type=jax.ShapeDtypeStruct((b, _D), jnp.float32),
        scratch_types=[
            pltpu.VMEM((bpw,), jnp.int32),
            pltpu.VMEM((bpw, _D), jnp.float32),
            pltpu.SemaphoreType.DMA,
        ],
    )
    def gk(table_hbm, idx_hbm, out_hbm, idx_v, rows_v, sem):
        wid = lax.axis_index("s") * nc + lax.axis_index("c")
        base = wid * bpw
        pltpu.sync_copy(idx_hbm.at[pl.ds(base, bpw)], idx_v)
        pltpu.async_copy(table_hbm.at[idx_v], rows_v, sem).wait()
        pltpu.sync_copy(rows_v, out_hbm.at[pl.ds(base, bpw)])

    return gk(emb, idx)


def kernel(z, embeddings):
    zf = z.reshape(-1, z.shape[-1])
    idx = _tc_argmin(zf, embeddings)
    zq = _sc_gather(embeddings, idx).reshape(z.shape)
    return zq, idx.reshape(z.shape[:-1])
